# baseline (speedup 1.0000x reference)
import functools

import jax
import jax.numpy as jnp
from jax import lax
from jax.experimental import pallas as pl
from jax.experimental.pallas import tpu as pltpu
from jax.experimental.pallas import tpu_sc as plsc

_L = 64
_B = 8
_LANES = 16
_NCHUNK = _L // _LANES
_CHUNK = _LANES * _B
_NREG = _CHUNK // _LANES


def _logf(x):
    ix = lax.bitcast_convert_type(x, jnp.int32)
    ix = ix + jnp.int32(0x3F800000 - 0x3F3504F3)
    k = lax.shift_right_arithmetic(ix, 23) - jnp.int32(0x7F)
    m = lax.bitcast_convert_type(
        (ix & jnp.int32(0x007FFFFF)) + jnp.int32(0x3F3504F3), jnp.float32)
    f = m - jnp.float32(1.0)
    s = f / (jnp.float32(2.0) + f)
    z = s * s
    w = z * z
    t1 = w * (jnp.float32(0.40000972152) + w * jnp.float32(0.24279078841))
    t2 = z * (jnp.float32(0.66666662693) + w * jnp.float32(0.28498786688))
    r = t2 + t1
    hfsq = jnp.float32(0.5) * f * f
    kf = k.astype(jnp.float32)
    return (s * (hfsq + r) + (kf * jnp.float32(9.0580006145e-06) - hfsq) + f
            + kf * jnp.float32(6.9313812256e-01))


def _shuf(v, p):
    return v.at[p].get(mode="promise_in_bounds")


def _sc_body(alpha_hbm, unif_hbm, arcs_hbm, lp_hbm, ent_hbm,
             a_v, u_v, arcs_v, lp_v, ent_v, sem_in, sem_out):
    wid = lax.axis_index("s")

    @pl.when(wid < _NCHUNK)
    def _():
        start = wid * _CHUNK
        cp_a = pltpu.async_copy(alpha_hbm.at[pl.ds(start, _CHUNK)], a_v, sem_in)
        cp_u = pltpu.async_copy(unif_hbm.at[pl.ds(start, _CHUNK)], u_v, sem_in)
        cp_a.wait()
        cp_u.wait()

        lane = lax.iota(jnp.int32, _LANES)
        br = lane & 7
        perms = [lane ^ s for s in (4, 2, 1)]
        pick = (lane & 1) << 3
        half = lax.shift_right_arithmetic(lane, 1)

        arcs_o = jnp.zeros((_LANES,), jnp.int32)
        lp_o = jnp.zeros((_LANES,), jnp.float32)
        ent_o = jnp.zeros((_LANES,), jnp.float32)
        for k in range(_NREG):
            a = a_v[pl.ds(k * _LANES, _LANES)]
            u = u_v[pl.ds(k * _LANES, _LANES)]

            score = a + (-_logf(-_logf(u)))
            idx = br
            for p in perms:
                o_s = _shuf(score, p)
                o_i = _shuf(idx, p)
                better = (o_s > score) | ((o_s == score) & (o_i < idx))
                score = jnp.where(better, o_s, score)
                idx = jnp.where(better, o_i, idx)

            amax = a
            for p in perms:
                amax = jnp.maximum(amax, _shuf(amax, p))
            e = jnp.exp(a - amax)
            ssum = e
            for p in perms:
                ssum = ssum + _shuf(ssum, p)
            lp = a - (amax + _logf(ssum))

            sel = jnp.where(br == idx, lp, jnp.float32(0.0))
            acc = e * lp
            for p in perms:
                sel = sel + _shuf(sel, p)
                acc = acc + _shuf(acc, p)
            ent = -acc / ssum

            m = half == k
            arcs_o = jnp.where(m, _shuf(idx, pick), arcs_o)
            lp_o = jnp.where(m, _shuf(sel, pick), lp_o)
            ent_o = jnp.where(m, _shuf(ent, pick), ent_o)

        arcs_v[...] = arcs_o
        lp_v[...] = lp_o
        ent_v[...] = ent_o
        base = wid * _LANES
        cp0 = pltpu.async_copy(arcs_v, arcs_hbm.at[pl.ds(base, _LANES)], sem_out)
        cp1 = pltpu.async_copy(lp_v, lp_hbm.at[pl.ds(base, _LANES)], sem_out)
        cp2 = pltpu.async_copy(ent_v, ent_hbm.at[pl.ds(base, _LANES)], sem_out)
        cp0.wait()
        cp1.wait()
        cp2.wait()


@functools.lru_cache(maxsize=None)
def _sc_call():
    return pl.kernel(
        _sc_body,
        out_type=(
            jax.ShapeDtypeStruct((_L,), jnp.int32),
            jax.ShapeDtypeStruct((_L,), jnp.float32),
            jax.ShapeDtypeStruct((_L,), jnp.float32),
        ),
        mesh=plsc.VectorSubcoreMesh(core_axis_name="c", subcore_axis_name="s",
                                    num_cores=1),
        scratch_types=[
            pltpu.VMEM((_CHUNK,), jnp.float32),
            pltpu.VMEM((_CHUNK,), jnp.float32),
            pltpu.VMEM((_LANES,), jnp.int32),
            pltpu.VMEM((_LANES,), jnp.float32),
            pltpu.VMEM((_LANES,), jnp.float32),
            pltpu.SemaphoreType.DMA,
            pltpu.SemaphoreType.DMA,
        ],
    )


def kernel(alpha, uniform):
    arcs, lp, ent = _sc_call()(alpha.reshape(_L * _B), uniform.reshape(_L * _B))
    return arcs[None, :], lp[None, :], ent[None, :]


# speedup vs baseline: 1.0085x; 1.0085x over previous
import functools

import jax
import jax.numpy as jnp
from jax import lax
from jax.experimental import pallas as pl
from jax.experimental.pallas import tpu as pltpu
from jax.experimental.pallas import tpu_sc as plsc

_L = 64
_B = 8
_LANES = 16
_NCHUNK = _L // _LANES


def _logf(x):
    ix = lax.bitcast_convert_type(x, jnp.int32)
    ix = ix + jnp.int32(0x3F800000 - 0x3F3504F3)
    k = lax.shift_right_arithmetic(ix, 23) - jnp.int32(0x7F)
    m = lax.bitcast_convert_type(
        (ix & jnp.int32(0x007FFFFF)) + jnp.int32(0x3F3504F3), jnp.float32)
    f = m - jnp.float32(1.0)
    s = f / (jnp.float32(2.0) + f)
    z = s * s
    w = z * z
    t1 = w * (jnp.float32(0.40000972152) + w * jnp.float32(0.24279078841))
    t2 = z * (jnp.float32(0.66666662693) + w * jnp.float32(0.28498786688))
    r = t2 + t1
    hfsq = jnp.float32(0.5) * f * f
    kf = k.astype(jnp.float32)
    return (s * (hfsq + r) + (kf * jnp.float32(9.0580006145e-06) - hfsq) + f
            + kf * jnp.float32(6.9313812256e-01))


def _sc_body(au_hbm, arcs_hbm, lp_hbm, ent_hbm,
             au_v, arcs_v, lp_v, ent_v, sem_in, sem_out):
    wid = lax.axis_index("s")

    @pl.when(wid < _NCHUNK)
    def _():
        pltpu.async_copy(au_hbm, au_v, sem_in).wait()

        base = wid * _LANES
        a = [au_v[0, b, pl.ds(base, _LANES)] for b in range(_B)]
        u = [au_v[1, b, pl.ds(base, _LANES)] for b in range(_B)]

        score = a[0] + (-_logf(-_logf(u[0])))
        idx = jnp.zeros((_LANES,), jnp.int32)
        for b in range(1, _B):
            sb = a[b] + (-_logf(-_logf(u[b])))
            upd = sb > score
            score = jnp.where(upd, sb, score)
            idx = jnp.where(upd, jnp.full((_LANES,), b, jnp.int32), idx)

        amax = a[0]
        for b in range(1, _B):
            amax = jnp.maximum(amax, a[b])
        e = [jnp.exp(a[b] - amax) for b in range(_B)]
        ssum = e[0]
        for b in range(1, _B):
            ssum = ssum + e[b]
        shift = amax + _logf(ssum)

        lp_sel = jnp.zeros((_LANES,), jnp.float32)
        acc = jnp.zeros((_LANES,), jnp.float32)
        for b in range(_B):
            lpb = a[b] - shift
            acc = acc + e[b] * lpb
            lp_sel = jnp.where(idx == b, lpb, lp_sel)

        arcs_v[...] = idx
        lp_v[...] = lp_sel
        ent_v[...] = -acc / ssum
        cp0 = pltpu.async_copy(arcs_v, arcs_hbm.at[pl.ds(base, _LANES)], sem_out)
        cp1 = pltpu.async_copy(lp_v, lp_hbm.at[pl.ds(base, _LANES)], sem_out)
        cp2 = pltpu.async_copy(ent_v, ent_hbm.at[pl.ds(base, _LANES)], sem_out)
        cp0.wait()
        cp1.wait()
        cp2.wait()


@functools.lru_cache(maxsize=None)
def _sc_call():
    return pl.kernel(
        _sc_body,
        out_type=(
            jax.ShapeDtypeStruct((_L,), jnp.int32),
            jax.ShapeDtypeStruct((_L,), jnp.float32),
            jax.ShapeDtypeStruct((_L,), jnp.float32),
        ),
        mesh=plsc.VectorSubcoreMesh(core_axis_name="c", subcore_axis_name="s",
                                    num_cores=1),
        scratch_types=[
            pltpu.VMEM((2, _B, _L), jnp.float32),
            pltpu.VMEM((_LANES,), jnp.int32),
            pltpu.VMEM((_LANES,), jnp.float32),
            pltpu.VMEM((_LANES,), jnp.float32),
            pltpu.SemaphoreType.DMA,
            pltpu.SemaphoreType.DMA,
        ],
    )


def kernel(alpha, uniform):
    au = jnp.stack([alpha.T, uniform.T])
    arcs, lp, ent = _sc_call()(au)
    return arcs[None, :], lp[None, :], ent[None, :]


# speedup vs baseline: 1.0184x; 1.0098x over previous
import functools

import jax
import jax.numpy as jnp
from jax import lax
from jax.experimental import pallas as pl
from jax.experimental.pallas import tpu as pltpu
from jax.experimental.pallas import tpu_sc as plsc

_L = 64
_B = 8
_LANES = 16
_NCHUNK = _L // _LANES


def _logf(x):
    ix = lax.bitcast_convert_type(x, jnp.int32)
    ix = ix + jnp.int32(0x3F800000 - 0x3F3504F3)
    k = lax.shift_right_arithmetic(ix, 23) - jnp.int32(0x7F)
    m = lax.bitcast_convert_type(
        (ix & jnp.int32(0x007FFFFF)) + jnp.int32(0x3F3504F3), jnp.float32)
    f = m - jnp.float32(1.0)
    s = f / (jnp.float32(2.0) + f)
    z = s * s
    w = z * z
    t1 = w * (jnp.float32(0.40000972152) + w * jnp.float32(0.24279078841))
    t2 = z * (jnp.float32(0.66666662693) + w * jnp.float32(0.28498786688))
    r = t2 + t1
    hfsq = jnp.float32(0.5) * f * f
    kf = k.astype(jnp.float32)
    return (s * (hfsq + r) + (kf * jnp.float32(9.0580006145e-06) - hfsq) + f
            + kf * jnp.float32(6.9313812256e-01))


def _sc_body(au_hbm, arcs_hbm, lp_hbm, ent_hbm,
             au_v, arcs_v, lp_v, ent_v, sem_in, sem_out):
    wid = lax.axis_index("s")

    @pl.when(wid < _NCHUNK)
    def _():
        pltpu.async_copy(au_hbm, au_v, sem_in).wait()

        base = wid * _LANES
        a = [au_v[0, b, pl.ds(base, _LANES)] for b in range(_B)]
        u = [au_v[1, b, pl.ds(base, _LANES)] for b in range(_B)]

        score = a[0] + (-_logf(-_logf(u[0])))
        idx = jnp.zeros((_LANES,), jnp.int32)
        for b in range(1, _B):
            sb = a[b] + (-_logf(-_logf(u[b])))
            upd = sb > score
            score = jnp.where(upd, sb, score)
            idx = jnp.where(upd, jnp.full((_LANES,), b, jnp.int32), idx)

        arcs_v[...] = idx
        cp0 = pltpu.async_copy(arcs_v, arcs_hbm.at[pl.ds(base, _LANES)], sem_out)

        amax = a[0]
        for b in range(1, _B):
            amax = jnp.maximum(amax, a[b])
        e = [jnp.exp(a[b] - amax) for b in range(_B)]
        ssum = e[0]
        for b in range(1, _B):
            ssum = ssum + e[b]
        shift = amax + _logf(ssum)

        lp_sel = jnp.zeros((_LANES,), jnp.float32)
        acc = jnp.zeros((_LANES,), jnp.float32)
        for b in range(_B):
            lpb = a[b] - shift
            acc = acc + e[b] * lpb
            lp_sel = jnp.where(idx == b, lpb, lp_sel)

        lp_v[...] = lp_sel
        cp1 = pltpu.async_copy(lp_v, lp_hbm.at[pl.ds(base, _LANES)], sem_out)
        ent_v[...] = -acc / ssum
        cp2 = pltpu.async_copy(ent_v, ent_hbm.at[pl.ds(base, _LANES)], sem_out)
        cp0.wait()
        cp1.wait()
        cp2.wait()


@functools.lru_cache(maxsize=None)
def _sc_call():
    return pl.kernel(
        _sc_body,
        out_type=(
            jax.ShapeDtypeStruct((_L,), jnp.int32),
            jax.ShapeDtypeStruct((_L,), jnp.float32),
            jax.ShapeDtypeStruct((_L,), jnp.float32),
        ),
        mesh=plsc.VectorSubcoreMesh(core_axis_name="c", subcore_axis_name="s",
                                    num_cores=1),
        scratch_types=[
            pltpu.VMEM((2, _B, _L), jnp.float32),
            pltpu.VMEM((_LANES,), jnp.int32),
            pltpu.VMEM((_LANES,), jnp.float32),
            pltpu.VMEM((_LANES,), jnp.float32),
            pltpu.SemaphoreType.DMA,
            pltpu.SemaphoreType.DMA,
        ],
    )


def kernel(alpha, uniform):
    au = jnp.stack([alpha.T, uniform.T])
    arcs, lp, ent = _sc_call()(au)
    return arcs[None, :], lp[None, :], ent[None, :]
